# baseline (speedup 1.0000x reference)
import functools

import jax
import jax.numpy as jnp
from jax import lax
from jax.experimental import pallas as pl
from jax.experimental.pallas import tpu as pltpu

_LANES = 256
_CHAINS = 4


def _gru_body(S_chunk, Bb, H, unroll,
              x_ref, lens_ref, wi0_ref, wi1_ref, w0_ref, w1_ref,
              brz_ref, bn_ref, bhn_ref,
              o_ref,
              gx0, gx1, ot, hc, xt):
    s = pl.program_id(1)
    sc = s - 1
    t0 = sc * S_chunk
    CH = Bb // _CHAINS
    TS = 4
    ROWS = TS * Bb
    NSLAB = S_chunk // TS
    brz = brz_ref[...]
    bn = bn_ref[...]

    TSG = 8
    NGRAN = S_chunk // TSG

    def xpose_granule(g, buf):
        xs = x_ref[:, pl.ds(TSG * g, TSG), :]
        xt[buf, pl.ds(TSG * g, TSG)] = pltpu.einshape("bte->tbe", xs)

    def _run_chunk():
        pltpu.matmul_push_rhs(wi0_ref[...], 0, 0)
        pltpu.matmul_push_rhs(wi1_ref[...], 0, 1)
        rbuf = sc % 2
        wbuf = 1 - rbuf

        def slab_lhs(j):
            return xt[rbuf, pl.ds(TS * j, TS)].reshape(ROWS, _LANES)

        lhs = slab_lhs(0)
        pltpu.matmul_acc_lhs(0, lhs, 0, load_staged_rhs=0)
        pltpu.matmul_acc_lhs(0, lhs, 1, load_staged_rhs=0)
        for j in range(NSLAB):
            a = (j % 4) * 64
            if j + 1 < NSLAB:
                nxt = slab_lhs(j + 1)
                an = ((j + 1) % 4) * 64
                pltpu.matmul_acc_lhs(an, nxt, 0)
                pltpu.matmul_acc_lhs(an, nxt, 1)
            p0 = pltpu.matmul_pop(a, (ROWS, _LANES), jnp.float32, 0)
            p1 = pltpu.matmul_pop(a, (ROWS, _LANES), jnp.float32, 1)
            gx0[pl.ds(TS * j, TS)] = (p0 + brz).reshape(TS, Bb, _LANES)
            gx1[pl.ds(TS * j, TS)] = (p1[:, :H] + bn).reshape(TS, Bb, H)
        _recur(wbuf)

    def _recur(wbuf):
        pltpu.matmul_push_rhs(w0_ref[...], 0, 0)
        pltpu.matmul_push_rhs(w1_ref[...], 0, 1)
        lens = lens_ref[...]
        lens_c = [lens[CH * c:CH * (c + 1), :] for c in range(_CHAINS)]
        ones = jnp.ones((CH, H), jnp.float32)

        hs = [hc[pl.ds(CH * c, CH), :] for c in range(_CHAINS)]

        def issue_acc(c, h, base, lsr=None):
            lhsh = jnp.concatenate([h, ones], axis=1)
            pltpu.matmul_acc_lhs(base + 4 * c, lhsh, 0, load_staged_rhs=lsr)
            pltpu.matmul_acc_lhs(base + 4 * c, lhsh, 1, load_staged_rhs=lsr)

        for c in range(_CHAINS):
            issue_acc(c, hs[c], 0, 0 if c == 0 else None)

        def step(t, par, hs):
            out = []
            base, nbase = 16 * par, 16 * (1 - par)
            for c in range(_CHAINS):
                grz = pltpu.matmul_pop(base + 4 * c, (CH, _LANES),
                                       jnp.float32, 0)
                gnw = pltpu.matmul_pop(base + 4 * c, (CH, _LANES),
                                       jnp.float32, 1)
                gi = gx0[t, pl.ds(CH * c, CH), :]
                gin = gx1[t, pl.ds(CH * c, CH), :]
                r = jax.nn.sigmoid(gi[:, :H] + grz[:, :H])
                z = jax.nn.sigmoid(gi[:, H:] + grz[:, H:])
                n = jnp.tanh(gin + r * gnw[:, :H])
                h_new = n + z * (hs[c] - n)
                issue_acc(c, h_new, nbase)
                valid = lens_c[c] > (t0 + t)
                ot[t, pl.ds(CH * c, CH), :] = jnp.where(valid, h_new, 0.0)
                out.append(h_new)
            return out

        def body(i, carry):
            hs = list(carry)
            for u in range(unroll):
                if u % 8 == 0:
                    xpose_granule(i * (unroll // 8) + u // 8, wbuf)
                hs = step(i * unroll + u, u % 2, hs)
            return tuple(hs)

        hs = list(lax.fori_loop(0, S_chunk // unroll, body, tuple(hs)))

        for c in range(_CHAINS):
            pltpu.matmul_pop(4 * c, (CH, _LANES), jnp.float32, 0)
            pltpu.matmul_pop(4 * c, (CH, _LANES), jnp.float32, 1)

        for c in range(_CHAINS):
            hc[pl.ds(CH * c, CH), :] = hs[c]

        o_ref[...] = pltpu.einshape("tbh->bth", ot[...])
    del bhn_ref

    @pl.when(s == 0)
    def _stage_first():
        hc[...] = jnp.zeros_like(hc)
        for g in range(NGRAN):
            xpose_granule(g, 0)

    @pl.when(s > 0)
    def _main():
        _run_chunk()


@functools.partial(jax.jit, static_argnames=())
def _gru_encoder(sents, lengths, wih, whh, bih, bhh):
    B, S, E = sents.shape
    H = whh.shape[-1]
    assert E == 256 and H == 128, "kernel tuned for E=256, H=128"
    Bb, S_chunk = 64, 128
    assert B % Bb == 0 and S % S_chunk == 0
    n_bblk, n_chunk = B // Bb, S // S_chunk
    f32 = jnp.float32

    wih_p = jnp.transpose(wih, (1, 0, 2)).reshape(E, 3 * H).astype(f32)
    whh_p = jnp.transpose(whh, (1, 0, 2)).reshape(H, 3 * H).astype(f32)
    wi0 = wih_p[:, :2 * H]
    wi1 = jnp.zeros((E, _LANES), f32).at[:, :H].set(wih_p[:, 2 * H:])
    w0 = jnp.zeros((_LANES, _LANES), f32).at[:H, :].set(whh_p[:, :2 * H])
    w1 = (jnp.zeros((_LANES, _LANES), f32)
          .at[:H, :H].set(whh_p[:, 2 * H:])
          .at[H:H + 1, :H].set(bhh[2].astype(f32)))
    brz = jnp.concatenate([bih[0] + bhh[0], bih[1] + bhh[1]], axis=1)
    bn = bih[2].astype(f32)
    bhn = bhh[2].astype(f32)
    lens2 = lengths.astype(jnp.int32).reshape(B, 1)

    body = functools.partial(_gru_body, S_chunk, Bb, H, 16)

    last = n_chunk - 1
    out = pl.pallas_call(
        body,
        out_shape=jax.ShapeDtypeStruct((B, S, H), f32),
        grid=(n_bblk, n_chunk + 1),
        in_specs=[
            pl.BlockSpec((Bb, S_chunk, E),
                         lambda i, s: (i, jnp.minimum(s, last), 0)),
            pl.BlockSpec((Bb, 1), lambda i, s: (i, 0)),
            pl.BlockSpec((E, _LANES), lambda i, s: (0, 0)),
            pl.BlockSpec((E, _LANES), lambda i, s: (0, 0)),
            pl.BlockSpec((_LANES, _LANES), lambda i, s: (0, 0)),
            pl.BlockSpec((_LANES, _LANES), lambda i, s: (0, 0)),
            pl.BlockSpec((1, _LANES), lambda i, s: (0, 0)),
            pl.BlockSpec((1, H), lambda i, s: (0, 0)),
            pl.BlockSpec((1, H), lambda i, s: (0, 0)),
        ],
        out_specs=pl.BlockSpec((Bb, S_chunk, H),
                               lambda i, s: (i, jnp.maximum(s - 1, 0), 0)),
        scratch_shapes=[
            pltpu.VMEM((S_chunk, Bb, _LANES), f32),
            pltpu.VMEM((S_chunk, Bb, H), f32),
            pltpu.VMEM((S_chunk, Bb, H), f32),
            pltpu.VMEM((Bb, H), f32),
            pltpu.VMEM((2, S_chunk, Bb, E), f32),
        ],
        compiler_params=pltpu.CompilerParams(
            dimension_semantics=("parallel", "arbitrary"),
            vmem_limit_bytes=62 * 1024 * 1024,
        ),
    )(sents.astype(f32), lens2, wi0, wi1, w0, w1, brz, bn, bhn)
    return out


def kernel(sents, lengths, wih, whh, bih, bhh):
    return _gru_encoder(sents, lengths, wih, whh, bih, bhh)


# speedup vs baseline: 1.0219x; 1.0219x over previous
import functools

import jax
import jax.numpy as jnp
from jax import lax
from jax.experimental import pallas as pl
from jax.experimental.pallas import tpu as pltpu

_LANES = 256
_CHAINS = 4


def _gru_body(S_chunk, Bb, H, unroll,
              x_ref, lens_ref, wi0_ref, wi1_ref, w0_ref, w1_ref,
              brz_ref, bn_ref, bhn_ref,
              o_ref,
              gx0, gx1, ot, hc):
    s = pl.program_id(1)
    t0 = s * S_chunk
    CH = Bb // _CHAINS

    @pl.when(s == 0)
    def _():
        hc[...] = jnp.zeros_like(hc)

    pltpu.matmul_push_rhs(wi0_ref[...], 0, 0)
    pltpu.matmul_push_rhs(wi1_ref[...], 0, 1)
    brz = brz_ref[...]
    bn = bn_ref[...]

    TS = 4
    ROWS = TS * Bb
    NSLAB = S_chunk // TS

    def slab_lhs(j):
        xs = x_ref[:, pl.ds(TS * j, TS), :]
        return pltpu.einshape("bte->tbe", xs).reshape(ROWS, _LANES)

    lhs = slab_lhs(0)
    pltpu.matmul_acc_lhs(0, lhs, 0, load_staged_rhs=0)
    pltpu.matmul_acc_lhs(0, lhs, 1, load_staged_rhs=0)
    for j in range(NSLAB):
        a = (j % 4) * 64
        if j + 1 < NSLAB:
            nxt = slab_lhs(j + 1)
            an = ((j + 1) % 4) * 64
            pltpu.matmul_acc_lhs(an, nxt, 0)
            pltpu.matmul_acc_lhs(an, nxt, 1)
        p0 = pltpu.matmul_pop(a, (ROWS, _LANES), jnp.float32, 0)
        p1 = pltpu.matmul_pop(a, (ROWS, _LANES), jnp.float32, 1)
        gx0[pl.ds(TS * j, TS)] = (p0 + brz).reshape(TS, Bb, _LANES)
        gx1[pl.ds(TS * j, TS)] = (p1[:, :H] + bn).reshape(TS, Bb, H)

    pltpu.matmul_push_rhs(w0_ref[...], 0, 0)
    pltpu.matmul_push_rhs(w1_ref[...], 0, 1)
    del bhn_ref
    lens = lens_ref[...]
    lens_c = [lens[CH * c:CH * (c + 1), :] for c in range(_CHAINS)]
    ones = jnp.ones((CH, H), jnp.float32)

    hs = [hc[pl.ds(CH * c, CH), :] for c in range(_CHAINS)]

    def issue_acc(c, h, base, lsr=None):
        lhsh = jnp.concatenate([h, ones], axis=1)
        pltpu.matmul_acc_lhs(base + 4 * c, lhsh, 0, load_staged_rhs=lsr)
        pltpu.matmul_acc_lhs(base + 4 * c, lhsh, 1, load_staged_rhs=lsr)

    for c in range(_CHAINS):
        issue_acc(c, hs[c], 0, 0 if c == 0 else None)

    def step(t, par, hs):
        out = []
        base, nbase = 16 * par, 16 * (1 - par)
        for c in range(_CHAINS):
            grz = pltpu.matmul_pop(base + 4 * c, (CH, _LANES), jnp.float32, 0)
            gnw = pltpu.matmul_pop(base + 4 * c, (CH, _LANES), jnp.float32, 1)
            gi = gx0[t, pl.ds(CH * c, CH), :]
            gin = gx1[t, pl.ds(CH * c, CH), :]
            r = jax.nn.sigmoid(gi[:, :H] + grz[:, :H])
            z = jax.nn.sigmoid(gi[:, H:] + grz[:, H:])
            n = jnp.tanh(gin + r * gnw[:, :H])
            h_new = n + z * (hs[c] - n)
            issue_acc(c, h_new, nbase)
            valid = lens_c[c] > (t0 + t)
            ot[t, pl.ds(CH * c, CH), :] = jnp.where(valid, h_new, 0.0)
            out.append(h_new)
        return out

    def body(i, carry):
        hs = list(carry)
        for u in range(unroll):
            hs = step(i * unroll + u, u % 2, hs)
        return tuple(hs)

    hs = list(lax.fori_loop(0, S_chunk // unroll, body, tuple(hs)))

    for c in range(_CHAINS):
        pltpu.matmul_pop(4 * c, (CH, _LANES), jnp.float32, 0)
        pltpu.matmul_pop(4 * c, (CH, _LANES), jnp.float32, 1)

    for c in range(_CHAINS):
        hc[pl.ds(CH * c, CH), :] = hs[c]

    o_ref[...] = pltpu.einshape("tbh->bth", ot[...])


@functools.partial(jax.jit, static_argnames=())
def _gru_encoder(sents, lengths, wih, whh, bih, bhh):
    B, S, E = sents.shape
    H = whh.shape[-1]
    assert E == 256 and H == 128, "kernel tuned for E=256, H=128"
    Bb, S_chunk = 64, 128
    assert B % Bb == 0 and S % S_chunk == 0
    n_bblk, n_chunk = B // Bb, S // S_chunk
    f32 = jnp.float32

    wih_p = jnp.transpose(wih, (1, 0, 2)).reshape(E, 3 * H).astype(f32)
    whh_p = jnp.transpose(whh, (1, 0, 2)).reshape(H, 3 * H).astype(f32)
    wi0 = wih_p[:, :2 * H]
    wi1 = jnp.zeros((E, _LANES), f32).at[:, :H].set(wih_p[:, 2 * H:])
    w0 = jnp.zeros((_LANES, _LANES), f32).at[:H, :].set(whh_p[:, :2 * H])
    w1 = (jnp.zeros((_LANES, _LANES), f32)
          .at[:H, :H].set(whh_p[:, 2 * H:])
          .at[H:H + 1, :H].set(bhh[2].astype(f32)))
    brz = jnp.concatenate([bih[0] + bhh[0], bih[1] + bhh[1]], axis=1)
    bn = bih[2].astype(f32)
    bhn = bhh[2].astype(f32)
    lens2 = lengths.astype(jnp.int32).reshape(B, 1)

    body = functools.partial(_gru_body, S_chunk, Bb, H, 16)

    out = pl.pallas_call(
        body,
        out_shape=jax.ShapeDtypeStruct((B, S, H), f32),
        grid=(n_bblk, n_chunk),
        in_specs=[
            pl.BlockSpec((Bb, S_chunk, E), lambda i, s: (i, s, 0)),
            pl.BlockSpec((Bb, 1), lambda i, s: (i, 0)),
            pl.BlockSpec((E, _LANES), lambda i, s: (0, 0)),
            pl.BlockSpec((E, _LANES), lambda i, s: (0, 0)),
            pl.BlockSpec((_LANES, _LANES), lambda i, s: (0, 0)),
            pl.BlockSpec((_LANES, _LANES), lambda i, s: (0, 0)),
            pl.BlockSpec((1, _LANES), lambda i, s: (0, 0)),
            pl.BlockSpec((1, H), lambda i, s: (0, 0)),
            pl.BlockSpec((1, H), lambda i, s: (0, 0)),
        ],
        out_specs=pl.BlockSpec((Bb, S_chunk, H), lambda i, s: (i, s, 0)),
        scratch_shapes=[
            pltpu.VMEM((S_chunk, Bb, _LANES), f32),
            pltpu.VMEM((S_chunk, Bb, H), f32),
            pltpu.VMEM((S_chunk, Bb, H), f32),
            pltpu.VMEM((Bb, H), f32),
        ],
        compiler_params=pltpu.CompilerParams(
            dimension_semantics=("parallel", "arbitrary"),
            vmem_limit_bytes=60 * 1024 * 1024,
        ),
    )(sents.astype(f32), lens2, wi0, wi1, w0, w1, brz, bn, bhn)
    return out


def kernel(sents, lengths, wih, whh, bih, bhh):
    return _gru_encoder(sents, lengths, wih, whh, bih, bhh)
